# Initial kernel scaffold; baseline (speedup 1.0000x reference)
import jax
import jax.numpy as jnp
from jax.experimental import pallas as pl


def kernel(x, edge_index, weights, W1, b1, W2, b2):
    raise NotImplementedError("write your pallas kernel here")


# speedup vs baseline: 8.0227x; 8.0227x over previous
import functools

import jax
import jax.numpy as jnp
from jax import lax
from jax.experimental import pallas as pl
from jax.experimental.pallas import tpu as pltpu
from jax.experimental.pallas import tpu_sc as plsc

N = 10000
E = 320000
D = 128
NC, NS = 2, 16
NW = NC * NS
EPW = E // NW
CHUNK = 80
NCHUNK = EPW // CHUNK
RPS = 624
RPS_LAST = N - (NS - 1) * RPS


def _sc_aggregate(x, edges_r, zeros):
    mesh = plsc.VectorSubcoreMesh(core_axis_name="c", subcore_axis_name="s")

    @functools.partial(
        pl.kernel,
        out_type=jax.ShapeDtypeStruct((NC, N, D), jnp.float32),
        mesh=mesh,
        scratch_types=[
            pltpu.VMEM((2, NCHUNK, CHUNK), jnp.int32),
            pltpu.VMEM((CHUNK, D), jnp.float32),
            pltpu.VMEM_SHARED((N, D), jnp.float32),
            pltpu.SemaphoreType.DMA,
        ],
    )
    def k(x_hbm, e_hbm, z_hbm, out_hbm, idx_v, rows_v, agg_sh, sem):
        c = lax.axis_index("c")
        s = lax.axis_index("s")
        wid = c * NS + s
        pltpu.sync_copy(e_hbm.at[wid], idx_v)
        r0 = s * RPS

        @pl.when(s < NS - 1)
        def _():
            pltpu.sync_copy(z_hbm.at[pl.ds(0, RPS)], agg_sh.at[pl.ds(r0, RPS)])

        @pl.when(s == NS - 1)
        def _():
            pltpu.sync_copy(
                z_hbm.at[pl.ds(0, RPS_LAST)],
                agg_sh.at[pl.ds((NS - 1) * RPS, RPS_LAST)],
            )

        plsc.subcore_barrier()

        def body(j, carry):
            pltpu.async_copy(x_hbm.at[idx_v.at[0, j]], rows_v, sem).wait()
            pltpu.sync_copy(rows_v, agg_sh.at[idx_v.at[1, j]], add=True)
            return carry

        lax.fori_loop(0, NCHUNK, body, 0)
        plsc.subcore_barrier()

        @pl.when(s < NS - 1)
        def _():
            pltpu.sync_copy(
                agg_sh.at[pl.ds(r0, RPS)], out_hbm.at[c, pl.ds(r0, RPS)]
            )

        @pl.when(s == NS - 1)
        def _():
            pltpu.sync_copy(
                agg_sh.at[pl.ds((NS - 1) * RPS, RPS_LAST)],
                out_hbm.at[c, pl.ds((NS - 1) * RPS, RPS_LAST)],
            )

    return k(x, edges_r, zeros)


def _tc_finish(x, parts, w2d, W1, b1, W2, b2):
    def body(x_ref, p_ref, w_ref, w1_ref, b1_ref, w2_ref, b2_ref, out_ref):
        h = x_ref[...] + p_ref[0] + p_ref[1]
        t = jnp.dot(h, w1_ref[...].T, preferred_element_type=jnp.float32)
        t = jnp.maximum(t + b1_ref[...], 0.0)
        wv = w_ref[...]
        v = jnp.sum(t * wv, axis=0, keepdims=True)
        sw = jnp.sum(wv)
        out = jnp.dot(v, w2_ref[...].T, preferred_element_type=jnp.float32)
        out_ref[...] = out + sw * b2_ref[...]

    return pl.pallas_call(
        body,
        out_shape=jax.ShapeDtypeStruct((1, D), jnp.float32),
    )(x, parts, w2d, W1, b1, W2, b2)


def kernel(x, edge_index, weights, W1, b1, W2, b2):
    edges_r = edge_index.reshape(2, NW, NCHUNK, CHUNK).transpose(1, 0, 2, 3)
    zeros = jnp.zeros((RPS_LAST, D), jnp.float32)
    parts = _sc_aggregate(x, edges_r, zeros)
    out = _tc_finish(x, parts, weights.reshape(N, 1), W1, b1, W2, b2)
    return out.reshape(1, 1, D)
